# baseline (speedup 1.0000x reference)
import jax
import jax.numpy as jnp
from jax import lax
from jax.experimental import pallas as pl
from jax.experimental.pallas import tpu as pltpu
from jax.experimental.pallas import tpu_sc as plsc

N = 10000
E = 320000
D = 128
H = 64

NC = 2
NS = 16
NW = NC * NS
EW = E // NW
C = 125
NCH = EW // C
NPAD = 10000
RT = NPAD // NS
CW = 8
G = 10


def _make_sc_segment_sum(with_counts: bool):
  mesh = plsc.VectorSubcoreMesh(core_axis_name="c", subcore_axis_name="s")

  out_type = [
      jax.ShapeDtypeStruct((NC * NPAD, H), jnp.float32),
      jax.ShapeDtypeStruct((NC * NPAD, H), jnp.float32),
  ]
  scratch = [
      pltpu.VMEM((G, C), jnp.int32),
      pltpu.VMEM((G, C), jnp.int32),
      pltpu.VMEM((C, H), jnp.float32),
      pltpu.VMEM((C, H), jnp.float32),
      pltpu.VMEM((C, H), jnp.float32),
      pltpu.VMEM((C, H), jnp.float32),
      pltpu.VMEM_SHARED((NPAD, H), jnp.float32),
      pltpu.VMEM_SHARED((NPAD, H), jnp.float32),
      pltpu.SemaphoreType.DMA,
      pltpu.SemaphoreType.DMA,
      pltpu.SemaphoreType.DMA,
      pltpu.SemaphoreType.DMA,
      pltpu.SemaphoreType.DMA,
      pltpu.SemaphoreType.DMA,
  ]
  if with_counts:
    out_type += [
        jax.ShapeDtypeStruct((NC * NPAD, CW), jnp.float32),
    ]
    scratch += [
        pltpu.VMEM((C, CW), jnp.float32),
        pltpu.VMEM((C, CW), jnp.float32),
        pltpu.VMEM_SHARED((NPAD, CW), jnp.float32),
    ]

  def body(u_f, u_b, src3, dst3, z64, z16, ones16, *rest):
    if with_counts:
      (s_f_out, s_b_out, c_out,
       idx_s, idx_d, buf_fa, buf_ba, buf_fb, buf_bb, acc_f, acc_b,
       sem_fa, sem_ba, sem_fb, sem_bb, sem_sa, sem_sb,
       oned_v, ones_v, acc_c) = rest
    else:
      (s_f_out, s_b_out,
       idx_s, idx_d, buf_fa, buf_ba, buf_fb, buf_bb, acc_f, acc_b,
       sem_fa, sem_ba, sem_fb, sem_bb, sem_sa, sem_sb) = rest

    cid = lax.axis_index("c")
    sid = lax.axis_index("s")
    wid = sid * NC + cid
    base = sid * RT

    pltpu.sync_copy(z64, acc_f.at[pl.ds(base, RT)])
    pltpu.sync_copy(z64, acc_b.at[pl.ds(base, RT)])
    if with_counts:
      pltpu.sync_copy(z16, acc_c.at[pl.ds(base, RT)])
      pltpu.sync_copy(ones16.at[0], oned_v)
      pltpu.sync_copy(ones16.at[1], ones_v)
    plsc.subcore_barrier()

    def issue(j, buf_f, buf_b, sem_f, sem_b):
      df = pltpu.async_copy(u_f.at[idx_s.at[j]], buf_f, sem_f)
      db = pltpu.async_copy(u_b.at[idx_d.at[j]], buf_b, sem_b)
      return df, db

    def scatter_async(j, buf_f, buf_b, ssem):
      out = [pltpu.async_copy(buf_f, acc_f.at[idx_d.at[j]], ssem, add=True),
             pltpu.async_copy(buf_b, acc_b.at[idx_s.at[j]], ssem, add=True)]
      if with_counts:
        out.append(
            pltpu.async_copy(oned_v, acc_c.at[idx_d.at[j]], ssem, add=True))
        out.append(
            pltpu.async_copy(ones_v, acc_c.at[idx_s.at[j]], ssem, add=True))
      return out

    def superchunk(g, carry):
      pltpu.sync_copy(src3.at[wid, pl.ds(g * G, G)], idx_s)
      pltpu.sync_copy(dst3.at[wid, pl.ds(g * G, G)], idx_d)

      def wait_gather(j, buf_f, buf_b, sem_f, sem_b):
        pltpu.make_async_copy(u_f.at[idx_s.at[j]], buf_f, sem_f).wait()
        pltpu.make_async_copy(u_b.at[idx_d.at[j]], buf_b, sem_b).wait()

      issue(0, buf_fa, buf_ba, sem_fa, sem_ba)

      def pair(k, c2):
        j0 = 2 * k
        j1 = j0 + 1
        db = issue(j1, buf_fb, buf_bb, sem_fb, sem_bb)
        wait_gather(j0, buf_fa, buf_ba, sem_fa, sem_ba)
        sa = scatter_async(j0, buf_fa, buf_ba, sem_sa)
        db[0].wait()
        db[1].wait()
        sb = scatter_async(j1, buf_fb, buf_bb, sem_sb)
        for d in sa:
          d.wait()

        @pl.when(k + 1 < G // 2)
        def _():
          issue(j0 + 2, buf_fa, buf_ba, sem_fa, sem_ba)

        for d in sb:
          d.wait()
        return c2

      lax.fori_loop(0, G // 2, pair, 0)
      return carry

    lax.fori_loop(0, NCH // G, superchunk, 0)
    plsc.subcore_barrier()

    obase = cid * NPAD + base
    pltpu.sync_copy(acc_f.at[pl.ds(base, RT)], s_f_out.at[pl.ds(obase, RT)])
    pltpu.sync_copy(acc_b.at[pl.ds(base, RT)], s_b_out.at[pl.ds(obase, RT)])
    if with_counts:
      pltpu.sync_copy(acc_c.at[pl.ds(base, RT)], c_out.at[pl.ds(obase, RT)])

  return pl.kernel(
      body, out_type=out_type, mesh=mesh, scratch_types=scratch,
      compiler_params=pltpu.CompilerParams(use_tc_tiling_on_sc=False))


_sc_layer1 = _make_sc_segment_sum(with_counts=True)
_sc_layer2 = _make_sc_segment_sum(with_counts=False)


def _tc_pre(x_ref, wlf, wlb, wrf, wrb, blf, blb, uf_ref, ub_ref, r_ref):
  xx = x_ref[...]
  uf_ref[...] = jnp.dot(xx, wlf[...], preferred_element_type=jnp.float32)
  ub_ref[...] = jnp.dot(xx, wlb[...], preferred_element_type=jnp.float32)
  r_ref[...] = (
      jnp.dot(xx, wrf[...] + wrb[...], preferred_element_type=jnp.float32)
      + blf[...] + blb[...])


BS = 1000
NB = N // BS


def _pre_block(sf0, sf1, sb0, sb1, cnt0, cnt1, r_ref):
  cd = cnt0[:, 0:1] + cnt1[:, 0:1]
  cs = cnt0[:, 1:2] + cnt1[:, 1:2]
  return ((sf0[...] + sf1[...]) / jnp.maximum(cd, 1.0)
          + (sb0[...] + sb1[...]) / jnp.maximum(cs, 1.0) + r_ref[...])


def _tc_mid(sf0, sf1, sb0, sb1, cnt0, cnt1, r_ref, g_ref, be_ref,
            wlf, wlb, wrf, wrb, blf, blb, uf_ref, ub_ref, r2_ref,
            pre_scr, st_scr):
  p = pl.program_id(0)
  b = pl.program_id(1)

  @pl.when(p == 0)
  def _():
    pre = _pre_block(sf0, sf1, sb0, sb1, cnt0, cnt1, r_ref)
    pre_scr[pl.ds(b * BS, BS), :] = pre

    @pl.when(b == 0)
    def _():
      st_scr[...] = jnp.zeros((2, H), jnp.float32)
    st_scr[0:1, :] += jnp.sum(pre, axis=0)[None, :]
    st_scr[1:2, :] += jnp.sum(pre * pre, axis=0)[None, :]

  @pl.when(p == 1)
  def _():
    m = st_scr[0:1, :] / N
    v = st_scr[1:2, :] / N - m * m
    pre = pre_scr[pl.ds(b * BS, BS), :]
    h = jnp.maximum(
        (pre - m) / jnp.sqrt(v + 1e-5) * g_ref[...] + be_ref[...], 0.0)
    uf_ref[...] = jnp.dot(h, wlf[...], preferred_element_type=jnp.float32)
    ub_ref[...] = jnp.dot(h, wlb[...], preferred_element_type=jnp.float32)
    r2_ref[...] = (
        jnp.dot(h, wrf[...] + wrb[...], preferred_element_type=jnp.float32)
        + blf[...] + blb[...])


def _tc_final(sf_ref, sb_ref, cnt_ref, r_ref, g_ref, be_ref, out_ref):
  cd = cnt_ref[0:N, 0:1] + cnt_ref[NPAD:NPAD + N, 0:1]
  cs = cnt_ref[0:N, 1:2] + cnt_ref[NPAD:NPAD + N, 1:2]
  s_f = sf_ref[0:N, :] + sf_ref[NPAD:NPAD + N, :]
  s_b = sb_ref[0:N, :] + sb_ref[NPAD:NPAD + N, :]
  pre = (s_f / jnp.maximum(cd, 1.0) + s_b / jnp.maximum(cs, 1.0) + r_ref[...])
  m = jnp.mean(pre, axis=0)
  v = jnp.mean((pre - m[None, :]) ** 2, axis=0)
  hb = ((pre - m[None, :]) / jnp.sqrt(v[None, :] + 1e-5) * g_ref[...]
        + be_ref[...])
  out_ref[...] = jnp.max(jnp.maximum(hb, 0.0), axis=0)[None, :]


def kernel(x, edge_index, Wl_f1, bl_f1, Wr_f1, Wl_b1, bl_b1, Wr_b1,
           Wl_f2, bl_f2, Wr_f2, Wl_b2, bl_b2, Wr_b2, g1, be1, g2, be2):
  src3 = edge_index[0].reshape(NW, NCH, C)
  dst3 = edge_index[1].reshape(NW, NCH, C)
  z64 = jnp.zeros((RT, H), jnp.float32)
  z16 = jnp.zeros((RT, CW), jnp.float32)
  eye2 = jnp.concatenate([jnp.eye(2, CW, dtype=jnp.float32)] * C, axis=0)
  ones16 = eye2.reshape(C, 2, CW).transpose(1, 0, 2)

  nh = jax.ShapeDtypeStruct((N, H), jnp.float32)
  u_f1, u_b1, r1 = pl.pallas_call(
      _tc_pre, out_shape=[nh, nh, nh])(x, Wl_f1, Wl_b1, Wr_f1, Wr_b1,
                                       bl_f1, bl_b1)

  sf1, sb1, cnt = _sc_layer1(u_f1, u_b1, src3, dst3, z64, z16, ones16)

  row_blk = pl.BlockSpec((BS, H), lambda p, b: (b, 0))
  hi_blk = pl.BlockSpec((BS, H), lambda p, b: (b + NB, 0))
  cnt_blk = pl.BlockSpec((BS, CW), lambda p, b: (b, 0))
  cnt_hi = pl.BlockSpec((BS, CW), lambda p, b: (b + NB, 0))
  full = lambda s: pl.BlockSpec(s, lambda p, b: (0, 0))
  u_f2, u_b2, r2 = pl.pallas_call(
      _tc_mid,
      grid=(2, NB),
      in_specs=[row_blk, hi_blk, row_blk, hi_blk, cnt_blk, cnt_hi, row_blk,
                full((1, H)), full((1, H)),
                full((H, H)), full((H, H)), full((H, H)), full((H, H)),
                full((1, H)), full((1, H))],
      out_specs=[row_blk, row_blk, row_blk],
      out_shape=[nh, nh, nh],
      scratch_shapes=[pltpu.VMEM((N, H), jnp.float32),
                      pltpu.VMEM((2, H), jnp.float32)],
  )(sf1, sf1, sb1, sb1, cnt, cnt, r1, g1.reshape(1, H), be1.reshape(1, H),
    Wl_f2, Wl_b2, Wr_f2, Wr_b2, bl_f2.reshape(1, H), bl_b2.reshape(1, H))

  sf2, sb2 = _sc_layer2(u_f2, u_b2, src3, dst3, z64, z16, ones16)

  out = pl.pallas_call(
      _tc_final, out_shape=jax.ShapeDtypeStruct((1, H), jnp.float32))(
          sf2, sb2, cnt, r2, g2, be2)
  return out.reshape(H)


# speedup vs baseline: 1.0266x; 1.0266x over previous
import jax
import jax.numpy as jnp
from jax import lax
from jax.experimental import pallas as pl
from jax.experimental.pallas import tpu as pltpu
from jax.experimental.pallas import tpu_sc as plsc

N = 10000
E = 320000
D = 128
H = 64

NC = 2
NS = 16
NW = NC * NS
EW = E // NW
C = 125
NCH = EW // C
NPAD = 10000
RT = NPAD // NS
CW = 8
G = 20


def _make_sc_segment_sum(with_counts: bool):
  mesh = plsc.VectorSubcoreMesh(core_axis_name="c", subcore_axis_name="s")

  out_type = [
      jax.ShapeDtypeStruct((NC * NPAD, H), jnp.float32),
      jax.ShapeDtypeStruct((NC * NPAD, H), jnp.float32),
  ]
  scratch = [
      pltpu.VMEM((G, C), jnp.int32),
      pltpu.VMEM((G, C), jnp.int32),
      pltpu.VMEM((C, H), jnp.float32),
      pltpu.VMEM((C, H), jnp.float32),
      pltpu.VMEM((C, H), jnp.float32),
      pltpu.VMEM((C, H), jnp.float32),
      pltpu.VMEM_SHARED((NPAD, H), jnp.float32),
      pltpu.VMEM_SHARED((NPAD, H), jnp.float32),
      pltpu.SemaphoreType.DMA,
      pltpu.SemaphoreType.DMA,
      pltpu.SemaphoreType.DMA,
      pltpu.SemaphoreType.DMA,
      pltpu.SemaphoreType.DMA,
      pltpu.SemaphoreType.DMA,
  ]
  if with_counts:
    out_type += [
        jax.ShapeDtypeStruct((NC * NPAD, CW), jnp.float32),
    ]
    scratch += [
        pltpu.VMEM((C, CW), jnp.float32),
        pltpu.VMEM((C, CW), jnp.float32),
        pltpu.VMEM_SHARED((NPAD, CW), jnp.float32),
    ]

  def body(u_f, u_b, src3, dst3, z64, z16, ones16, *rest):
    if with_counts:
      (s_f_out, s_b_out, c_out,
       idx_s, idx_d, buf_fa, buf_ba, buf_fb, buf_bb, acc_f, acc_b,
       sem_fa, sem_ba, sem_fb, sem_bb, sem_sa, sem_sb,
       oned_v, ones_v, acc_c) = rest
    else:
      (s_f_out, s_b_out,
       idx_s, idx_d, buf_fa, buf_ba, buf_fb, buf_bb, acc_f, acc_b,
       sem_fa, sem_ba, sem_fb, sem_bb, sem_sa, sem_sb) = rest

    cid = lax.axis_index("c")
    sid = lax.axis_index("s")
    wid = sid * NC + cid
    base = sid * RT

    pltpu.sync_copy(z64, acc_f.at[pl.ds(base, RT)])
    pltpu.sync_copy(z64, acc_b.at[pl.ds(base, RT)])
    if with_counts:
      pltpu.sync_copy(z16, acc_c.at[pl.ds(base, RT)])
      pltpu.sync_copy(ones16.at[0], oned_v)
      pltpu.sync_copy(ones16.at[1], ones_v)
    plsc.subcore_barrier()

    def issue(j, buf_f, buf_b, sem_f, sem_b):
      df = pltpu.async_copy(u_f.at[idx_s.at[j]], buf_f, sem_f)
      db = pltpu.async_copy(u_b.at[idx_d.at[j]], buf_b, sem_b)
      return df, db

    def scatter_async(j, buf_f, buf_b, ssem):
      out = [pltpu.async_copy(buf_f, acc_f.at[idx_d.at[j]], ssem, add=True),
             pltpu.async_copy(buf_b, acc_b.at[idx_s.at[j]], ssem, add=True)]
      if with_counts:
        out.append(
            pltpu.async_copy(oned_v, acc_c.at[idx_d.at[j]], ssem, add=True))
        out.append(
            pltpu.async_copy(ones_v, acc_c.at[idx_s.at[j]], ssem, add=True))
      return out

    def superchunk(g, carry):
      pltpu.sync_copy(src3.at[wid, pl.ds(g * G, G)], idx_s)
      pltpu.sync_copy(dst3.at[wid, pl.ds(g * G, G)], idx_d)

      def wait_gather(j, buf_f, buf_b, sem_f, sem_b):
        pltpu.make_async_copy(u_f.at[idx_s.at[j]], buf_f, sem_f).wait()
        pltpu.make_async_copy(u_b.at[idx_d.at[j]], buf_b, sem_b).wait()

      issue(0, buf_fa, buf_ba, sem_fa, sem_ba)

      def pair(k, c2):
        j0 = 2 * k
        j1 = j0 + 1
        db = issue(j1, buf_fb, buf_bb, sem_fb, sem_bb)
        wait_gather(j0, buf_fa, buf_ba, sem_fa, sem_ba)
        sa = scatter_async(j0, buf_fa, buf_ba, sem_sa)
        db[0].wait()
        db[1].wait()
        sb = scatter_async(j1, buf_fb, buf_bb, sem_sb)
        for d in sa:
          d.wait()

        @pl.when(k + 1 < G // 2)
        def _():
          issue(j0 + 2, buf_fa, buf_ba, sem_fa, sem_ba)

        for d in sb:
          d.wait()
        return c2

      lax.fori_loop(0, G // 2, pair, 0)
      return carry

    lax.fori_loop(0, NCH // G, superchunk, 0)
    plsc.subcore_barrier()

    obase = cid * NPAD + base
    pltpu.sync_copy(acc_f.at[pl.ds(base, RT)], s_f_out.at[pl.ds(obase, RT)])
    pltpu.sync_copy(acc_b.at[pl.ds(base, RT)], s_b_out.at[pl.ds(obase, RT)])
    if with_counts:
      pltpu.sync_copy(acc_c.at[pl.ds(base, RT)], c_out.at[pl.ds(obase, RT)])

  return pl.kernel(
      body, out_type=out_type, mesh=mesh, scratch_types=scratch,
      compiler_params=pltpu.CompilerParams(use_tc_tiling_on_sc=False))


_sc_layer1 = _make_sc_segment_sum(with_counts=True)
_sc_layer2 = _make_sc_segment_sum(with_counts=False)


def _tc_pre(x_ref, wlf, wlb, wrf, wrb, blf, blb, uf_ref, ub_ref, r_ref):
  xx = x_ref[...]
  uf_ref[...] = jnp.dot(xx, wlf[...], preferred_element_type=jnp.float32)
  ub_ref[...] = jnp.dot(xx, wlb[...], preferred_element_type=jnp.float32)
  r_ref[...] = (
      jnp.dot(xx, wrf[...] + wrb[...], preferred_element_type=jnp.float32)
      + blf[...] + blb[...])


BS = 1000
NB = N // BS


def _pre_block(sf0, sf1, sb0, sb1, cnt0, cnt1, r_ref):
  cd = cnt0[:, 0:1] + cnt1[:, 0:1]
  cs = cnt0[:, 1:2] + cnt1[:, 1:2]
  return ((sf0[...] + sf1[...]) / jnp.maximum(cd, 1.0)
          + (sb0[...] + sb1[...]) / jnp.maximum(cs, 1.0) + r_ref[...])


def _tc_mid(sf0, sf1, sb0, sb1, cnt0, cnt1, r_ref, g_ref, be_ref,
            wlf, wlb, wrf, wrb, blf, blb, uf_ref, ub_ref, r2_ref,
            pre_scr, st_scr):
  p = pl.program_id(0)
  b = pl.program_id(1)

  @pl.when(p == 0)
  def _():
    pre = _pre_block(sf0, sf1, sb0, sb1, cnt0, cnt1, r_ref)
    pre_scr[pl.ds(b * BS, BS), :] = pre

    @pl.when(b == 0)
    def _():
      st_scr[...] = jnp.zeros((2, H), jnp.float32)
    st_scr[0:1, :] += jnp.sum(pre, axis=0)[None, :]
    st_scr[1:2, :] += jnp.sum(pre * pre, axis=0)[None, :]

  @pl.when(p == 1)
  def _():
    m = st_scr[0:1, :] / N
    v = st_scr[1:2, :] / N - m * m
    pre = pre_scr[pl.ds(b * BS, BS), :]
    h = jnp.maximum(
        (pre - m) / jnp.sqrt(v + 1e-5) * g_ref[...] + be_ref[...], 0.0)
    uf_ref[...] = jnp.dot(h, wlf[...], preferred_element_type=jnp.float32)
    ub_ref[...] = jnp.dot(h, wlb[...], preferred_element_type=jnp.float32)
    r2_ref[...] = (
        jnp.dot(h, wrf[...] + wrb[...], preferred_element_type=jnp.float32)
        + blf[...] + blb[...])


def _tc_final(sf_ref, sb_ref, cnt_ref, r_ref, g_ref, be_ref, out_ref):
  cd = cnt_ref[0:N, 0:1] + cnt_ref[NPAD:NPAD + N, 0:1]
  cs = cnt_ref[0:N, 1:2] + cnt_ref[NPAD:NPAD + N, 1:2]
  s_f = sf_ref[0:N, :] + sf_ref[NPAD:NPAD + N, :]
  s_b = sb_ref[0:N, :] + sb_ref[NPAD:NPAD + N, :]
  pre = (s_f / jnp.maximum(cd, 1.0) + s_b / jnp.maximum(cs, 1.0) + r_ref[...])
  m = jnp.mean(pre, axis=0)
  v = jnp.mean((pre - m[None, :]) ** 2, axis=0)
  hb = ((pre - m[None, :]) / jnp.sqrt(v[None, :] + 1e-5) * g_ref[...]
        + be_ref[...])
  out_ref[...] = jnp.max(jnp.maximum(hb, 0.0), axis=0)[None, :]


def kernel(x, edge_index, Wl_f1, bl_f1, Wr_f1, Wl_b1, bl_b1, Wr_b1,
           Wl_f2, bl_f2, Wr_f2, Wl_b2, bl_b2, Wr_b2, g1, be1, g2, be2):
  src3 = edge_index[0].reshape(NW, NCH, C)
  dst3 = edge_index[1].reshape(NW, NCH, C)
  z64 = jnp.zeros((RT, H), jnp.float32)
  z16 = jnp.zeros((RT, CW), jnp.float32)
  eye2 = jnp.concatenate([jnp.eye(2, CW, dtype=jnp.float32)] * C, axis=0)
  ones16 = eye2.reshape(C, 2, CW).transpose(1, 0, 2)

  nh = jax.ShapeDtypeStruct((N, H), jnp.float32)
  u_f1, u_b1, r1 = pl.pallas_call(
      _tc_pre, out_shape=[nh, nh, nh])(x, Wl_f1, Wl_b1, Wr_f1, Wr_b1,
                                       bl_f1, bl_b1)

  sf1, sb1, cnt = _sc_layer1(u_f1, u_b1, src3, dst3, z64, z16, ones16)

  row_blk = pl.BlockSpec((BS, H), lambda p, b: (b, 0))
  hi_blk = pl.BlockSpec((BS, H), lambda p, b: (b + NB, 0))
  cnt_blk = pl.BlockSpec((BS, CW), lambda p, b: (b, 0))
  cnt_hi = pl.BlockSpec((BS, CW), lambda p, b: (b + NB, 0))
  full = lambda s: pl.BlockSpec(s, lambda p, b: (0, 0))
  u_f2, u_b2, r2 = pl.pallas_call(
      _tc_mid,
      grid=(2, NB),
      in_specs=[row_blk, hi_blk, row_blk, hi_blk, cnt_blk, cnt_hi, row_blk,
                full((1, H)), full((1, H)),
                full((H, H)), full((H, H)), full((H, H)), full((H, H)),
                full((1, H)), full((1, H))],
      out_specs=[row_blk, row_blk, row_blk],
      out_shape=[nh, nh, nh],
      scratch_shapes=[pltpu.VMEM((N, H), jnp.float32),
                      pltpu.VMEM((2, H), jnp.float32)],
  )(sf1, sf1, sb1, sb1, cnt, cnt, r1, g1.reshape(1, H), be1.reshape(1, H),
    Wl_f2, Wl_b2, Wr_f2, Wr_b2, bl_f2.reshape(1, H), bl_b2.reshape(1, H))

  sf2, sb2 = _sc_layer2(u_f2, u_b2, src3, dst3, z64, z16, ones16)

  out = pl.pallas_call(
      _tc_final, out_shape=jax.ShapeDtypeStruct((1, H), jnp.float32))(
          sf2, sb2, cnt, r2, g2, be2)
  return out.reshape(H)
